# baseline (speedup 1.0000x reference)
import functools

import jax
import jax.numpy as jnp
from jax import lax
from jax.experimental import pallas as pl
from jax.experimental.pallas import tpu as pltpu
from jax.experimental.pallas import tpu_sc as plsc

_B = 8
_NW = 32


def _fps_call(xyzT, npoint):
    _, b, n = xyzT.shape

    def body(xyzT_ref, cent_ref):
        x = xyzT_ref[0]
        y = xyzT_ref[1]
        z = xyzT_ref[2]
        iota_n = lax.broadcasted_iota(jnp.int32, (b, n), 1)
        iota_p = lax.broadcasted_iota(jnp.int32, (b, npoint), 1)

        def step(i, carry):
            dists, far, ax, ay, az = carry
            oh = iota_n == far
            cx = jnp.sum(jnp.where(oh, x, 0.0), axis=1, keepdims=True)
            cy = jnp.sum(jnp.where(oh, y, 0.0), axis=1, keepdims=True)
            cz = jnp.sum(jnp.where(oh, z, 0.0), axis=1, keepdims=True)
            sel = iota_p == i
            ax = jnp.where(sel, cx, ax)
            ay = jnp.where(sel, cy, ay)
            az = jnp.where(sel, cz, az)
            dx = x - cx
            dy = y - cy
            dz = z - cz
            d = dx * dx + dy * dy + dz * dz
            dists = jnp.minimum(dists, d)
            m = jnp.max(dists, axis=1, keepdims=True)
            cand = jnp.where(dists == m, iota_n, n)
            far = jnp.min(cand, axis=1, keepdims=True)
            return dists, far, ax, ay, az

        init = (
            jnp.full((b, n), 1e10, jnp.float32),
            jnp.zeros((b, 1), jnp.int32),
            jnp.zeros((b, npoint), jnp.float32),
            jnp.zeros((b, npoint), jnp.float32),
            jnp.zeros((b, npoint), jnp.float32),
        )
        _, _, ax, ay, az = lax.fori_loop(0, npoint, step, init)
        cent_ref[0] = ax
        cent_ref[1] = ay
        cent_ref[2] = az

    return pl.pallas_call(
        body,
        out_shape=jax.ShapeDtypeStruct((3, b, npoint), jnp.float32),
    )(xyzT)


def _ball_gather_level(xyzT, centT, table, radius, K, P, N, C):
    tpb = _NW // _B
    PB = P // tpb
    nchunks = N // 16
    r2 = radius * radius
    G = max(1, 128 // K)
    GK = G * K
    ngroups = PB // G
    mesh = plsc.VectorSubcoreMesh(core_axis_name="c", subcore_axis_name="s")

    @functools.partial(
        pl.kernel,
        mesh=mesh,
        compiler_params=pltpu.CompilerParams(needs_layout_passes=False,
                                             use_tc_tiling_on_sc=False),
        out_type=jax.ShapeDtypeStruct((_B * P * K, C), jnp.float32),
        scratch_types=[
            pltpu.VMEM((N,), jnp.float32),
            pltpu.VMEM((N,), jnp.float32),
            pltpu.VMEM((N,), jnp.float32),
            pltpu.VMEM((PB,), jnp.float32),
            pltpu.VMEM((PB,), jnp.float32),
            pltpu.VMEM((PB,), jnp.float32),
            pltpu.VMEM((K + 16,), jnp.int32),
            pltpu.VMEM((PB * K,), jnp.int32),
            pltpu.VMEM((GK, C), jnp.float32),
            pltpu.VMEM((GK, C), jnp.float32),
            pltpu.SMEM((1,), jnp.int32),
            pltpu.SemaphoreType.DMA,
            pltpu.SemaphoreType.DMA,
        ],
    )
    def bq(xyzT_hbm, centT_hbm, table_hbm, out_hbm,
           xb, yb, zb, cxb, cyb, czb, raw, idx_all, rows0, rows1,
           cnt_s, sem0, sem1):
        wid = lax.axis_index("s") * 2 + lax.axis_index("c")
        bi = wid // tpb
        p0 = (wid % tpb) * PB
        pltpu.sync_copy(xyzT_hbm.at[0, bi], xb)
        pltpu.sync_copy(xyzT_hbm.at[1, bi], yb)
        pltpu.sync_copy(xyzT_hbm.at[2, bi], zb)
        pltpu.sync_copy(centT_hbm.at[0, bi, pl.ds(p0, PB)], cxb)
        pltpu.sync_copy(centT_hbm.at[1, bi, pl.ds(p0, PB)], cyb)
        pltpu.sync_copy(centT_hbm.at[2, bi, pl.ds(p0, PB)], czb)
        lane = lax.iota(jnp.int32, 16)
        base_row = bi * N

        def per_chunk(cc, carry):
            cxv = cxb[pl.ds(cc * 16, 16)]
            cyv = cyb[pl.ds(cc * 16, 16)]
            czv = czb[pl.ds(cc * 16, 16)]
            for j in range(16):
                cx = cxv[j]
                cy = cyv[j]
                cz = czv[j]
                cnt_s[0] = 0

                def chunk(ch, carry2):
                    cnt = cnt_s[0]

                    @pl.when(cnt < K)
                    def _():
                        off = ch * 16
                        dx = xb[pl.ds(off, 16)] - cx
                        dy = yb[pl.ds(off, 16)] - cy
                        dz = zb[pl.ds(off, 16)] - cz
                        d = dx * dx + dy * dy + dz * dz
                        m = d <= r2
                        pop = plsc.all_reduce_population_count(m)[0]

                        @pl.when(pop > 0)
                        def _():
                            key = jnp.where(m, lane + off, 0x40000000 + lane)
                            ks, _ = plsc.sort_key_val(key, key)
                            raw[pl.ds(jnp.minimum(cnt, K), 16)] = ks

                        cnt_s[0] = cnt + pop
                    return carry2

                lax.fori_loop(0, nchunks, chunk, 0)
                cnt = cnt_s[0]
                first = raw[pl.ds(0, 16)][0]
                ci = cc * 16 + j
                for q in range(K // 16):
                    v = raw[pl.ds(q * 16, 16)]
                    v = jnp.where(lane + (q * 16) < cnt, v, first)
                    idx_all[pl.ds(ci * K + q * 16, 16)] = v + base_row
            return carry

        lax.fori_loop(0, PB // 16, per_chunk, 0)

        out_base = (bi * P + p0) * K

        def fire(g, rows, sem):
            pltpu.async_copy(
                table_hbm.at[idx_all.at[pl.ds(g * GK, GK)]], rows, sem)

        def drain(g, rows, sem):
            pltpu.make_async_copy(
                table_hbm.at[idx_all.at[pl.ds(g * GK, GK)]], rows, sem).wait()
            pltpu.sync_copy(rows, out_hbm.at[pl.ds(out_base + g * GK, GK)])

        def gloop(g, carry):
            even = g % 2 == 0

            @pl.when(even)
            def _():
                fire(g, rows0, sem0)

            @pl.when(jnp.logical_not(even))
            def _():
                fire(g, rows1, sem1)

            @pl.when(jnp.logical_and(g > 0, even))
            def _():
                drain(g - 1, rows1, sem1)

            @pl.when(jnp.logical_and(g > 0, jnp.logical_not(even)))
            def _():
                drain(g - 1, rows0, sem0)
            return carry

        lax.fori_loop(0, ngroups, gloop, 0)
        if ngroups % 2 == 1:
            drain(ngroups - 1, rows0, sem0)
        else:
            drain(ngroups - 1, rows1, sem1)

    return bq(xyzT, centT, table)


def _mlp_max_call(grouped, cent_cols, w1, b1, w2, b2, w3, b3, K, CB):
    _, C = grouped.shape
    BP = cent_cols.shape[0]
    C1 = w1.shape[1]
    C3 = w3.shape[1]

    def body(g_ref, c_ref, w1_ref, b1_ref, w2_ref, b2_ref,
             w3_ref, b3_ref, o_ref):
        X = g_ref[...].reshape(CB, K, C) - c_ref[...][:, None, :]
        h = jnp.dot(X.reshape(CB * K, C), w1_ref[...],
                    preferred_element_type=jnp.float32)
        h = jnp.maximum(h + b1_ref[...], 0.0)
        h = jnp.maximum(
            jnp.dot(h, w2_ref[...], preferred_element_type=jnp.float32)
            + b2_ref[...], 0.0)
        h = jnp.maximum(
            jnp.dot(h, w3_ref[...], preferred_element_type=jnp.float32)
            + b3_ref[...], 0.0)
        o_ref[...] = jnp.max(h.reshape(CB, K, C3), axis=1)

    rep = lambda shape: pl.BlockSpec(shape, lambda i: (0, 0))
    return pl.pallas_call(
        body,
        grid=(BP // CB,),
        in_specs=[
            pl.BlockSpec((CB * K, C), lambda i: (i, 0)),
            pl.BlockSpec((CB, C), lambda i: (i, 0)),
            rep(w1.shape), rep(b1.shape),
            rep(w2.shape), rep(b2.shape), rep(w3.shape), rep(b3.shape),
        ],
        out_specs=pl.BlockSpec((CB, C3), lambda i: (i, 0)),
        out_shape=jax.ShapeDtypeStruct((BP, C3), jnp.float32),
    )(grouped, cent_cols, w1, b1, w2, b2, w3, b3)


def _sa3_call(tbl, w1, b1, w2, b2, w3, b3, P):
    _, C = tbl.shape
    C3 = w3.shape[1]

    def body(x_ref, w1_ref, b1_ref, w2_ref, b2_ref, w3_ref, b3_ref, o_ref):
        h = jnp.maximum(
            jnp.dot(x_ref[...], w1_ref[...], preferred_element_type=jnp.float32)
            + b1_ref[...], 0.0)
        h = jnp.maximum(
            jnp.dot(h, w2_ref[...], preferred_element_type=jnp.float32)
            + b2_ref[...], 0.0)
        h = jnp.maximum(
            jnp.dot(h, w3_ref[...], preferred_element_type=jnp.float32)
            + b3_ref[...], 0.0)
        o_ref[...] = jnp.max(h, axis=0, keepdims=True)[None]

    rep = lambda shape: pl.BlockSpec(shape, lambda i: (0, 0))
    out = pl.pallas_call(
        body,
        grid=(_B,),
        in_specs=[
            pl.BlockSpec((P, C), lambda i: (i, 0)),
            rep(w1.shape), rep(b1.shape), rep(w2.shape), rep(b2.shape),
            rep(w3.shape), rep(b3.shape),
        ],
        out_specs=pl.BlockSpec((1, 1, C3), lambda i: (i, 0, 0)),
        out_shape=jax.ShapeDtypeStruct((_B, 1, C3), jnp.float32),
    )(tbl, w1, b1, w2, b2, w3, b3)
    return out.reshape(_B, C3)


def _head_call(x, w1, b1, g1, be1, w2, b2, g2, be2, w3, b3):
    def body(x_ref, w1_ref, b1_ref, g1_ref, be1_ref, w2_ref, b2_ref,
             g2_ref, be2_ref, w3_ref, b3_ref, o_ref):
        h = jnp.maximum(
            jnp.dot(x_ref[...], w1_ref[...], preferred_element_type=jnp.float32)
            + b1_ref[...], 0.0)
        mu = jnp.mean(h, axis=0, keepdims=True)
        var = jnp.mean((h - mu) ** 2, axis=0, keepdims=True)
        h = g1_ref[...] * (h - mu) / jnp.sqrt(var + 1e-5) + be1_ref[...]
        h = jnp.maximum(
            jnp.dot(h, w2_ref[...], preferred_element_type=jnp.float32)
            + b2_ref[...], 0.0)
        mu = jnp.mean(h, axis=0, keepdims=True)
        var = jnp.mean((h - mu) ** 2, axis=0, keepdims=True)
        h = g2_ref[...] * (h - mu) / jnp.sqrt(var + 1e-5) + be2_ref[...]
        o_ref[...] = (
            jnp.dot(h, w3_ref[...], preferred_element_type=jnp.float32)
            + b3_ref[...])

    return pl.pallas_call(
        body,
        out_shape=jax.ShapeDtypeStruct((x.shape[0], w3.shape[1]), jnp.float32),
    )(x, w1, b1, g1, be1, w2, b2, g2, be2, w3, b3)


_SA1_RADII, _SA1_NSAMPLE = [0.1, 0.2, 0.4], [16, 32, 128]
_SA2_RADII, _SA2_NSAMPLE = [0.2, 0.4, 0.8], [32, 64, 128]


def _level_weights(params, prefix, i, c_feat, c_pad):
    w0 = params[f"{prefix}_r{i}_w0"]
    c1 = w0.shape[1]
    w1 = jnp.zeros((c_pad, c1), jnp.float32).at[: c_feat + 3].set(w0)
    b1 = params[f"{prefix}_r{i}_b0"].reshape(1, -1)
    w2 = params[f"{prefix}_r{i}_w1"]
    b2 = params[f"{prefix}_r{i}_b1"].reshape(1, -1)
    w3 = params[f"{prefix}_r{i}_w2"]
    b3 = params[f"{prefix}_r{i}_b2"].reshape(1, -1)
    return w1, b1, w2, b2, w3, b3


def kernel(xyz, points, params):
    B, N1, _ = xyz.shape
    P1, P2 = 512, 128

    t1 = jnp.concatenate([points, xyz], axis=-1)
    t1 = jnp.pad(t1, ((0, 0), (0, 0), (0, 10))).reshape(B * N1, 16)
    xyzT = jnp.transpose(xyz, (2, 0, 1))

    cent1 = _fps_call(xyzT, P1)
    c1rows = jnp.transpose(cent1, (1, 2, 0)).reshape(B * P1, 3)
    c1cols = jnp.pad(c1rows, ((0, 0), (3, 10)))

    parts = []
    for i, (r, K) in enumerate(zip(_SA1_RADII, _SA1_NSAMPLE)):
        grouped = _ball_gather_level(xyzT, cent1, t1, r, K, P1, N1, 16)
        w1, b1, w2, b2, w3, b3 = _level_weights(params, "sa1", i, 3, 16)
        parts.append(_mlp_max_call(grouped, c1cols, w1, b1, w2, b2,
                                   w3, b3, K, CB=64))
    feat1 = jnp.concatenate(parts, axis=-1)

    t2 = jnp.pad(jnp.concatenate([feat1, c1rows], axis=-1), ((0, 0), (0, 13)))

    cent2 = _fps_call(cent1, P2)
    c2rows = jnp.transpose(cent2, (1, 2, 0)).reshape(B * P2, 3)
    c2cols = jnp.pad(c2rows, ((0, 0), (320, 13)))

    parts = []
    for i, (r, K) in enumerate(zip(_SA2_RADII, _SA2_NSAMPLE)):
        grouped = _ball_gather_level(cent1, cent2, t2, r, K, P2, P1, 336)
        w1, b1, w2, b2, w3, b3 = _level_weights(params, "sa2", i, 320, 336)
        parts.append(_mlp_max_call(grouped, c2cols, w1, b1, w2, b2,
                                   w3, b3, K, CB=16))
    feat2 = jnp.concatenate(parts, axis=-1)

    t3 = jnp.concatenate([feat2, c2rows], axis=-1)
    feat3 = _sa3_call(
        t3,
        params["sa3_w0"], params["sa3_b0"].reshape(1, -1),
        params["sa3_w1"], params["sa3_b1"].reshape(1, -1),
        params["sa3_w2"], params["sa3_b2"].reshape(1, -1),
        P2)

    return _head_call(
        feat3,
        params["dn1_w"], params["dn1_b"].reshape(1, -1),
        params["bn1_gamma"].reshape(1, -1), params["bn1_beta"].reshape(1, -1),
        params["dn2_w"], params["dn2_b"].reshape(1, -1),
        params["bn2_gamma"].reshape(1, -1), params["bn2_beta"].reshape(1, -1),
        params["dn3_w"], params["dn3_b"].reshape(1, -1))


# speedup vs baseline: 1.2382x; 1.2382x over previous
import functools

import jax
import jax.numpy as jnp
from jax import lax
from jax.experimental import pallas as pl
from jax.experimental.pallas import tpu as pltpu
from jax.experimental.pallas import tpu_sc as plsc

_B = 8
_NW = 32


def _fps_call(xyzT, npoint):
    _, b, n = xyzT.shape

    def body(xyzT_ref, cent_ref):
        x = xyzT_ref[0]
        y = xyzT_ref[1]
        z = xyzT_ref[2]
        iota_n = lax.broadcasted_iota(jnp.int32, (b, n), 1)
        iota_p = lax.broadcasted_iota(jnp.int32, (b, npoint), 1)

        def step(i, carry):
            dists, far, ax, ay, az = carry
            oh = iota_n == far
            cx = jnp.sum(jnp.where(oh, x, 0.0), axis=1, keepdims=True)
            cy = jnp.sum(jnp.where(oh, y, 0.0), axis=1, keepdims=True)
            cz = jnp.sum(jnp.where(oh, z, 0.0), axis=1, keepdims=True)
            sel = iota_p == i
            ax = jnp.where(sel, cx, ax)
            ay = jnp.where(sel, cy, ay)
            az = jnp.where(sel, cz, az)
            dx = x - cx
            dy = y - cy
            dz = z - cz
            d = dx * dx + dy * dy + dz * dz
            dists = jnp.minimum(dists, d)
            m = jnp.max(dists, axis=1, keepdims=True)
            cand = jnp.where(dists == m, iota_n, n)
            far = jnp.min(cand, axis=1, keepdims=True)
            return dists, far, ax, ay, az

        init = (
            jnp.full((b, n), 1e10, jnp.float32),
            jnp.zeros((b, 1), jnp.int32),
            jnp.zeros((b, npoint), jnp.float32),
            jnp.zeros((b, npoint), jnp.float32),
            jnp.zeros((b, npoint), jnp.float32),
        )
        _, _, ax, ay, az = lax.fori_loop(0, npoint, step, init)
        cent_ref[0] = ax
        cent_ref[1] = ay
        cent_ref[2] = az

    return pl.pallas_call(
        body,
        out_shape=jax.ShapeDtypeStruct((3, b, npoint), jnp.float32),
    )(xyzT)


def _ball_gather_level(xyzT, centT, table, radius, K, P, N, C):
    tpb = _NW // _B
    PB = P // tpb
    nchunks = N // 16
    r2 = radius * radius
    G = max(1, 128 // K)
    GK = G * K
    ngroups = PB // G
    mesh = plsc.VectorSubcoreMesh(core_axis_name="c", subcore_axis_name="s")

    @functools.partial(
        pl.kernel,
        mesh=mesh,
        compiler_params=pltpu.CompilerParams(needs_layout_passes=False,
                                             use_tc_tiling_on_sc=False),
        out_type=jax.ShapeDtypeStruct((_B * P * K, C), jnp.float32),
        scratch_types=[
            pltpu.VMEM((N,), jnp.float32),
            pltpu.VMEM((N,), jnp.float32),
            pltpu.VMEM((N,), jnp.float32),
            pltpu.VMEM((PB,), jnp.float32),
            pltpu.VMEM((PB,), jnp.float32),
            pltpu.VMEM((PB,), jnp.float32),
            pltpu.VMEM((K + 16,), jnp.int32),
            pltpu.VMEM((PB * K,), jnp.int32),
            pltpu.VMEM((GK, C), jnp.float32),
            pltpu.VMEM((GK, C), jnp.float32),
            pltpu.SMEM((1,), jnp.int32),
            pltpu.SemaphoreType.DMA,
            pltpu.SemaphoreType.DMA,
        ],
    )
    def bq(xyzT_hbm, centT_hbm, table_hbm, out_hbm,
           xb, yb, zb, cxb, cyb, czb, raw, idx_all, rows0, rows1,
           cnt_s, sem0, sem1):
        wid = lax.axis_index("s") * 2 + lax.axis_index("c")
        bi = wid // tpb
        p0 = (wid % tpb) * PB
        pltpu.sync_copy(xyzT_hbm.at[0, bi], xb)
        pltpu.sync_copy(xyzT_hbm.at[1, bi], yb)
        pltpu.sync_copy(xyzT_hbm.at[2, bi], zb)
        pltpu.sync_copy(centT_hbm.at[0, bi, pl.ds(p0, PB)], cxb)
        pltpu.sync_copy(centT_hbm.at[1, bi, pl.ds(p0, PB)], cyb)
        pltpu.sync_copy(centT_hbm.at[2, bi, pl.ds(p0, PB)], czb)
        lane = lax.iota(jnp.int32, 16)
        base_row = bi * N

        def per_chunk(cc, carry):
            cxv = cxb[pl.ds(cc * 16, 16)]
            cyv = cyb[pl.ds(cc * 16, 16)]
            czv = czb[pl.ds(cc * 16, 16)]
            for j in range(16):
                cx = cxv[j]
                cy = cyv[j]
                cz = czv[j]

                def chunk(ch, cnt):
                    off = ch * 16
                    dx = xb[pl.ds(off, 16)] - cx
                    dy = yb[pl.ds(off, 16)] - cy
                    dz = zb[pl.ds(off, 16)] - cz
                    d = dx * dx + dy * dy + dz * dz
                    m = d <= r2
                    key = jnp.where(m, lane + off, 0x40000000 + lane)
                    ks, _ = plsc.sort_key_val(key, key)
                    raw[pl.ds(jnp.minimum(cnt, K), 16)] = ks
                    return cnt + plsc.all_reduce_population_count(m)[0]

                cnt = lax.fori_loop(0, nchunks, chunk, jnp.int32(0))
                first = raw[pl.ds(0, 16)][0]
                ci = cc * 16 + j
                for q in range(K // 16):
                    v = raw[pl.ds(q * 16, 16)]
                    v = jnp.where(lane + (q * 16) < cnt, v, first)
                    idx_all[pl.ds(ci * K + q * 16, 16)] = v + base_row
            return carry

        lax.fori_loop(0, PB // 16, per_chunk, 0)

        out_base = (bi * P + p0) * K

        def fire(g, rows, sem):
            pltpu.async_copy(
                table_hbm.at[idx_all.at[pl.ds(g * GK, GK)]], rows, sem)

        def drain(g, rows, sem):
            pltpu.make_async_copy(
                table_hbm.at[idx_all.at[pl.ds(g * GK, GK)]], rows, sem).wait()
            pltpu.sync_copy(rows, out_hbm.at[pl.ds(out_base + g * GK, GK)])

        def gloop(g, carry):
            even = g % 2 == 0

            @pl.when(even)
            def _():
                fire(g, rows0, sem0)

            @pl.when(jnp.logical_not(even))
            def _():
                fire(g, rows1, sem1)

            @pl.when(jnp.logical_and(g > 0, even))
            def _():
                drain(g - 1, rows1, sem1)

            @pl.when(jnp.logical_and(g > 0, jnp.logical_not(even)))
            def _():
                drain(g - 1, rows0, sem0)
            return carry

        lax.fori_loop(0, ngroups, gloop, 0)
        if ngroups % 2 == 1:
            drain(ngroups - 1, rows0, sem0)
        else:
            drain(ngroups - 1, rows1, sem1)

    return bq(xyzT, centT, table)


def _mlp_max_call(grouped, cent_cols, w1, b1, w2, b2, w3, b3, K, CB):
    _, C = grouped.shape
    BP = cent_cols.shape[0]
    C1 = w1.shape[1]
    C3 = w3.shape[1]

    def body(g_ref, c_ref, w1_ref, b1_ref, w2_ref, b2_ref,
             w3_ref, b3_ref, o_ref):
        X = g_ref[...].reshape(CB, K, C) - c_ref[...][:, None, :]
        h = jnp.dot(X.reshape(CB * K, C), w1_ref[...],
                    preferred_element_type=jnp.float32)
        h = jnp.maximum(h + b1_ref[...], 0.0)
        h = jnp.maximum(
            jnp.dot(h, w2_ref[...], preferred_element_type=jnp.float32)
            + b2_ref[...], 0.0)
        h = jnp.maximum(
            jnp.dot(h, w3_ref[...], preferred_element_type=jnp.float32)
            + b3_ref[...], 0.0)
        o_ref[...] = jnp.max(h.reshape(CB, K, C3), axis=1)

    rep = lambda shape: pl.BlockSpec(shape, lambda i: (0, 0))
    return pl.pallas_call(
        body,
        grid=(BP // CB,),
        in_specs=[
            pl.BlockSpec((CB * K, C), lambda i: (i, 0)),
            pl.BlockSpec((CB, C), lambda i: (i, 0)),
            rep(w1.shape), rep(b1.shape),
            rep(w2.shape), rep(b2.shape), rep(w3.shape), rep(b3.shape),
        ],
        out_specs=pl.BlockSpec((CB, C3), lambda i: (i, 0)),
        out_shape=jax.ShapeDtypeStruct((BP, C3), jnp.float32),
    )(grouped, cent_cols, w1, b1, w2, b2, w3, b3)


def _sa3_call(tbl, w1, b1, w2, b2, w3, b3, P):
    _, C = tbl.shape
    C3 = w3.shape[1]

    def body(x_ref, w1_ref, b1_ref, w2_ref, b2_ref, w3_ref, b3_ref, o_ref):
        h = jnp.maximum(
            jnp.dot(x_ref[...], w1_ref[...], preferred_element_type=jnp.float32)
            + b1_ref[...], 0.0)
        h = jnp.maximum(
            jnp.dot(h, w2_ref[...], preferred_element_type=jnp.float32)
            + b2_ref[...], 0.0)
        h = jnp.maximum(
            jnp.dot(h, w3_ref[...], preferred_element_type=jnp.float32)
            + b3_ref[...], 0.0)
        o_ref[...] = jnp.max(h, axis=0, keepdims=True)[None]

    rep = lambda shape: pl.BlockSpec(shape, lambda i: (0, 0))
    out = pl.pallas_call(
        body,
        grid=(_B,),
        in_specs=[
            pl.BlockSpec((P, C), lambda i: (i, 0)),
            rep(w1.shape), rep(b1.shape), rep(w2.shape), rep(b2.shape),
            rep(w3.shape), rep(b3.shape),
        ],
        out_specs=pl.BlockSpec((1, 1, C3), lambda i: (i, 0, 0)),
        out_shape=jax.ShapeDtypeStruct((_B, 1, C3), jnp.float32),
    )(tbl, w1, b1, w2, b2, w3, b3)
    return out.reshape(_B, C3)


def _head_call(x, w1, b1, g1, be1, w2, b2, g2, be2, w3, b3):
    def body(x_ref, w1_ref, b1_ref, g1_ref, be1_ref, w2_ref, b2_ref,
             g2_ref, be2_ref, w3_ref, b3_ref, o_ref):
        h = jnp.maximum(
            jnp.dot(x_ref[...], w1_ref[...], preferred_element_type=jnp.float32)
            + b1_ref[...], 0.0)
        mu = jnp.mean(h, axis=0, keepdims=True)
        var = jnp.mean((h - mu) ** 2, axis=0, keepdims=True)
        h = g1_ref[...] * (h - mu) / jnp.sqrt(var + 1e-5) + be1_ref[...]
        h = jnp.maximum(
            jnp.dot(h, w2_ref[...], preferred_element_type=jnp.float32)
            + b2_ref[...], 0.0)
        mu = jnp.mean(h, axis=0, keepdims=True)
        var = jnp.mean((h - mu) ** 2, axis=0, keepdims=True)
        h = g2_ref[...] * (h - mu) / jnp.sqrt(var + 1e-5) + be2_ref[...]
        o_ref[...] = (
            jnp.dot(h, w3_ref[...], preferred_element_type=jnp.float32)
            + b3_ref[...])

    return pl.pallas_call(
        body,
        out_shape=jax.ShapeDtypeStruct((x.shape[0], w3.shape[1]), jnp.float32),
    )(x, w1, b1, g1, be1, w2, b2, g2, be2, w3, b3)


_SA1_RADII, _SA1_NSAMPLE = [0.1, 0.2, 0.4], [16, 32, 128]
_SA2_RADII, _SA2_NSAMPLE = [0.2, 0.4, 0.8], [32, 64, 128]


def _level_weights(params, prefix, i, c_feat, c_pad):
    w0 = params[f"{prefix}_r{i}_w0"]
    c1 = w0.shape[1]
    w1 = jnp.zeros((c_pad, c1), jnp.float32).at[: c_feat + 3].set(w0)
    b1 = params[f"{prefix}_r{i}_b0"].reshape(1, -1)
    w2 = params[f"{prefix}_r{i}_w1"]
    b2 = params[f"{prefix}_r{i}_b1"].reshape(1, -1)
    w3 = params[f"{prefix}_r{i}_w2"]
    b3 = params[f"{prefix}_r{i}_b2"].reshape(1, -1)
    return w1, b1, w2, b2, w3, b3


def kernel(xyz, points, params):
    B, N1, _ = xyz.shape
    P1, P2 = 512, 128

    t1 = jnp.concatenate([points, xyz], axis=-1)
    t1 = jnp.pad(t1, ((0, 0), (0, 0), (0, 10))).reshape(B * N1, 16)
    xyzT = jnp.transpose(xyz, (2, 0, 1))

    cent1 = _fps_call(xyzT, P1)
    c1rows = jnp.transpose(cent1, (1, 2, 0)).reshape(B * P1, 3)
    c1cols = jnp.pad(c1rows, ((0, 0), (3, 10)))

    parts = []
    for i, (r, K) in enumerate(zip(_SA1_RADII, _SA1_NSAMPLE)):
        grouped = _ball_gather_level(xyzT, cent1, t1, r, K, P1, N1, 16)
        w1, b1, w2, b2, w3, b3 = _level_weights(params, "sa1", i, 3, 16)
        parts.append(_mlp_max_call(grouped, c1cols, w1, b1, w2, b2,
                                   w3, b3, K, CB=64))
    feat1 = jnp.concatenate(parts, axis=-1)

    t2 = jnp.pad(jnp.concatenate([feat1, c1rows], axis=-1), ((0, 0), (0, 13)))

    cent2 = _fps_call(cent1, P2)
    c2rows = jnp.transpose(cent2, (1, 2, 0)).reshape(B * P2, 3)
    c2cols = jnp.pad(c2rows, ((0, 0), (320, 13)))

    parts = []
    for i, (r, K) in enumerate(zip(_SA2_RADII, _SA2_NSAMPLE)):
        grouped = _ball_gather_level(cent1, cent2, t2, r, K, P2, P1, 336)
        w1, b1, w2, b2, w3, b3 = _level_weights(params, "sa2", i, 320, 336)
        parts.append(_mlp_max_call(grouped, c2cols, w1, b1, w2, b2,
                                   w3, b3, K, CB=16))
    feat2 = jnp.concatenate(parts, axis=-1)

    t3 = jnp.concatenate([feat2, c2rows], axis=-1)
    feat3 = _sa3_call(
        t3,
        params["sa3_w0"], params["sa3_b0"].reshape(1, -1),
        params["sa3_w1"], params["sa3_b1"].reshape(1, -1),
        params["sa3_w2"], params["sa3_b2"].reshape(1, -1),
        P2)

    return _head_call(
        feat3,
        params["dn1_w"], params["dn1_b"].reshape(1, -1),
        params["bn1_gamma"].reshape(1, -1), params["bn1_beta"].reshape(1, -1),
        params["dn2_w"], params["dn2_b"].reshape(1, -1),
        params["bn2_gamma"].reshape(1, -1), params["bn2_beta"].reshape(1, -1),
        params["dn3_w"], params["dn3_b"].reshape(1, -1))


# speedup vs baseline: 1.5614x; 1.2610x over previous
import functools

import jax
import jax.numpy as jnp
from jax import lax
from jax.experimental import pallas as pl
from jax.experimental.pallas import tpu as pltpu
from jax.experimental.pallas import tpu_sc as plsc

_B = 8
_NW = 32


def _fps_call(xyzT, npoint):
    _, b, n = xyzT.shape

    def body(xyzT_ref, cent_ref):
        x = xyzT_ref[0]
        y = xyzT_ref[1]
        z = xyzT_ref[2]
        iota_n = lax.broadcasted_iota(jnp.int32, (b, n), 1)
        iota_p = lax.broadcasted_iota(jnp.int32, (b, npoint), 1)

        def step(i, carry):
            dists, far, ax, ay, az = carry
            oh = iota_n == far
            cx = jnp.sum(jnp.where(oh, x, 0.0), axis=1, keepdims=True)
            cy = jnp.sum(jnp.where(oh, y, 0.0), axis=1, keepdims=True)
            cz = jnp.sum(jnp.where(oh, z, 0.0), axis=1, keepdims=True)
            sel = iota_p == i
            ax = jnp.where(sel, cx, ax)
            ay = jnp.where(sel, cy, ay)
            az = jnp.where(sel, cz, az)
            dx = x - cx
            dy = y - cy
            dz = z - cz
            d = dx * dx + dy * dy + dz * dz
            dists = jnp.minimum(dists, d)
            m = jnp.max(dists, axis=1, keepdims=True)
            cand = jnp.where(dists == m, iota_n, n)
            far = jnp.min(cand, axis=1, keepdims=True)
            return dists, far, ax, ay, az

        init = (
            jnp.full((b, n), 1e10, jnp.float32),
            jnp.zeros((b, 1), jnp.int32),
            jnp.zeros((b, npoint), jnp.float32),
            jnp.zeros((b, npoint), jnp.float32),
            jnp.zeros((b, npoint), jnp.float32),
        )
        _, _, ax, ay, az = lax.fori_loop(0, npoint, step, init)
        cent_ref[0] = ax
        cent_ref[1] = ay
        cent_ref[2] = az

    return pl.pallas_call(
        body,
        out_shape=jax.ShapeDtypeStruct((3, b, npoint), jnp.float32),
    )(xyzT)


def _ball_gather_msg(xyzT, centT, table, radii, Ks, P, N, C):
    tpb = _NW // _B
    PB = P // tpb
    nchunks = N // 16
    NR = len(Ks)
    r2s = [r * r for r in radii]
    Gs = [max(1, 128 // K) for K in Ks]
    GKs = [G * K for G, K in zip(Gs, Ks)]
    mesh = plsc.VectorSubcoreMesh(core_axis_name="c", subcore_axis_name="s")

    @functools.partial(
        pl.kernel,
        mesh=mesh,
        compiler_params=pltpu.CompilerParams(needs_layout_passes=False,
                                             use_tc_tiling_on_sc=False),
        out_type=tuple(jax.ShapeDtypeStruct((_B * P * K, C), jnp.float32)
                       for K in Ks),
        scratch_types=[
            pltpu.VMEM((N,), jnp.float32),
            pltpu.VMEM((N,), jnp.float32),
            pltpu.VMEM((N,), jnp.float32),
            pltpu.VMEM((PB,), jnp.float32),
            pltpu.VMEM((PB,), jnp.float32),
            pltpu.VMEM((PB,), jnp.float32),
            *[pltpu.VMEM((K + 16,), jnp.int32) for K in Ks],
            *[pltpu.VMEM((PB * K,), jnp.int32) for K in Ks],
            pltpu.VMEM((max(GKs), C), jnp.float32),
            pltpu.VMEM((max(GKs), C), jnp.float32),
            pltpu.SemaphoreType.DMA,
            pltpu.SemaphoreType.DMA,
        ],
    )
    def bq(xyzT_hbm, centT_hbm, table_hbm, *refs):
        outs = refs[:NR]
        xb, yb, zb, cxb, cyb, czb = refs[NR:NR + 6]
        raws = refs[NR + 6:NR + 6 + NR]
        idxs = refs[NR + 6 + NR:NR + 6 + 2 * NR]
        rows0, rows1, sem0, sem1 = refs[NR + 6 + 2 * NR:]
        wid = lax.axis_index("s") * 2 + lax.axis_index("c")
        bi = wid // tpb
        p0 = (wid % tpb) * PB
        pltpu.sync_copy(xyzT_hbm.at[0, bi], xb)
        pltpu.sync_copy(xyzT_hbm.at[1, bi], yb)
        pltpu.sync_copy(xyzT_hbm.at[2, bi], zb)
        pltpu.sync_copy(centT_hbm.at[0, bi, pl.ds(p0, PB)], cxb)
        pltpu.sync_copy(centT_hbm.at[1, bi, pl.ds(p0, PB)], cyb)
        pltpu.sync_copy(centT_hbm.at[2, bi, pl.ds(p0, PB)], czb)
        lane = lax.iota(jnp.int32, 16)
        base_row = bi * N

        def per_chunk(cc, carry):
            cxv = cxb[pl.ds(cc * 16, 16)]
            cyv = cyb[pl.ds(cc * 16, 16)]
            czv = czb[pl.ds(cc * 16, 16)]
            for j in range(16):
                cx = cxv[j]
                cy = cyv[j]
                cz = czv[j]

                def chunk(ch, cnts):
                    off = ch * 16
                    dx = xb[pl.ds(off, 16)] - cx
                    dy = yb[pl.ds(off, 16)] - cy
                    dz = zb[pl.ds(off, 16)] - cz
                    d = dx * dx + dy * dy + dz * dz
                    new = []
                    for i in range(NR):
                        m = d <= r2s[i]
                        key = jnp.where(m, lane + off, 0x40000000 + lane)
                        ks, _ = plsc.sort_key_val(key, key)
                        raws[i][pl.ds(jnp.minimum(cnts[i], Ks[i]), 16)] = ks
                        new.append(
                            cnts[i] + plsc.all_reduce_population_count(m)[0])
                    return tuple(new)

                cnts = lax.fori_loop(0, nchunks, chunk,
                                     (jnp.int32(0),) * NR)
                ci = cc * 16 + j
                for i in range(NR):
                    first = raws[i][pl.ds(0, 16)][0]
                    for q in range(Ks[i] // 16):
                        v = raws[i][pl.ds(q * 16, 16)]
                        v = jnp.where(lane + (q * 16) < cnts[i], v, first)
                        idxs[i][pl.ds(ci * Ks[i] + q * 16, 16)] = v + base_row
            return carry

        lax.fori_loop(0, PB // 16, per_chunk, 0)

        for i in range(NR):
            K, GK = Ks[i], GKs[i]
            ngroups = PB // Gs[i]
            idx_all = idxs[i]
            out_hbm = outs[i]
            out_base = (bi * P + p0) * K

            def fire(g, rows, sem):
                pltpu.async_copy(
                    table_hbm.at[idx_all.at[pl.ds(g * GK, GK)]],
                    rows.at[pl.ds(0, GK)], sem)

            def drain(g, rows, sem):
                pltpu.make_async_copy(
                    table_hbm.at[idx_all.at[pl.ds(g * GK, GK)]],
                    rows.at[pl.ds(0, GK)], sem).wait()
                pltpu.sync_copy(rows.at[pl.ds(0, GK)],
                                out_hbm.at[pl.ds(out_base + g * GK, GK)])

            def gloop(g, carry):
                even = g % 2 == 0

                @pl.when(even)
                def _():
                    fire(g, rows0, sem0)

                @pl.when(jnp.logical_not(even))
                def _():
                    fire(g, rows1, sem1)

                @pl.when(jnp.logical_and(g > 0, even))
                def _():
                    drain(g - 1, rows1, sem1)

                @pl.when(jnp.logical_and(g > 0, jnp.logical_not(even)))
                def _():
                    drain(g - 1, rows0, sem0)
                return carry

            lax.fori_loop(0, ngroups, gloop, 0)
            if ngroups % 2 == 1:
                drain(ngroups - 1, rows0, sem0)
            else:
                drain(ngroups - 1, rows1, sem1)

    return bq(xyzT, centT, table)


def _mlp_max_call(grouped, cent_cols, w1, b1, w2, b2, w3, b3, K, CB):
    _, C = grouped.shape
    BP = cent_cols.shape[0]
    C1 = w1.shape[1]
    C3 = w3.shape[1]

    def body(g_ref, c_ref, w1_ref, b1_ref, w2_ref, b2_ref,
             w3_ref, b3_ref, o_ref):
        X = g_ref[...].reshape(CB, K, C) - c_ref[...][:, None, :]
        h = jnp.dot(X.reshape(CB * K, C), w1_ref[...],
                    preferred_element_type=jnp.float32)
        h = jnp.maximum(h + b1_ref[...], 0.0)
        h = jnp.maximum(
            jnp.dot(h, w2_ref[...], preferred_element_type=jnp.float32)
            + b2_ref[...], 0.0)
        h = jnp.maximum(
            jnp.dot(h, w3_ref[...], preferred_element_type=jnp.float32)
            + b3_ref[...], 0.0)
        o_ref[...] = jnp.max(h.reshape(CB, K, C3), axis=1)

    rep = lambda shape: pl.BlockSpec(shape, lambda i: (0, 0))
    return pl.pallas_call(
        body,
        grid=(BP // CB,),
        in_specs=[
            pl.BlockSpec((CB * K, C), lambda i: (i, 0)),
            pl.BlockSpec((CB, C), lambda i: (i, 0)),
            rep(w1.shape), rep(b1.shape),
            rep(w2.shape), rep(b2.shape), rep(w3.shape), rep(b3.shape),
        ],
        out_specs=pl.BlockSpec((CB, C3), lambda i: (i, 0)),
        out_shape=jax.ShapeDtypeStruct((BP, C3), jnp.float32),
    )(grouped, cent_cols, w1, b1, w2, b2, w3, b3)


def _sa3_call(tbl, w1, b1, w2, b2, w3, b3, P):
    _, C = tbl.shape
    C3 = w3.shape[1]

    def body(x_ref, w1_ref, b1_ref, w2_ref, b2_ref, w3_ref, b3_ref, o_ref):
        h = jnp.maximum(
            jnp.dot(x_ref[...], w1_ref[...], preferred_element_type=jnp.float32)
            + b1_ref[...], 0.0)
        h = jnp.maximum(
            jnp.dot(h, w2_ref[...], preferred_element_type=jnp.float32)
            + b2_ref[...], 0.0)
        h = jnp.maximum(
            jnp.dot(h, w3_ref[...], preferred_element_type=jnp.float32)
            + b3_ref[...], 0.0)
        o_ref[...] = jnp.max(h, axis=0, keepdims=True)[None]

    rep = lambda shape: pl.BlockSpec(shape, lambda i: (0, 0))
    out = pl.pallas_call(
        body,
        grid=(_B,),
        in_specs=[
            pl.BlockSpec((P, C), lambda i: (i, 0)),
            rep(w1.shape), rep(b1.shape), rep(w2.shape), rep(b2.shape),
            rep(w3.shape), rep(b3.shape),
        ],
        out_specs=pl.BlockSpec((1, 1, C3), lambda i: (i, 0, 0)),
        out_shape=jax.ShapeDtypeStruct((_B, 1, C3), jnp.float32),
    )(tbl, w1, b1, w2, b2, w3, b3)
    return out.reshape(_B, C3)


def _head_call(x, w1, b1, g1, be1, w2, b2, g2, be2, w3, b3):
    def body(x_ref, w1_ref, b1_ref, g1_ref, be1_ref, w2_ref, b2_ref,
             g2_ref, be2_ref, w3_ref, b3_ref, o_ref):
        h = jnp.maximum(
            jnp.dot(x_ref[...], w1_ref[...], preferred_element_type=jnp.float32)
            + b1_ref[...], 0.0)
        mu = jnp.mean(h, axis=0, keepdims=True)
        var = jnp.mean((h - mu) ** 2, axis=0, keepdims=True)
        h = g1_ref[...] * (h - mu) / jnp.sqrt(var + 1e-5) + be1_ref[...]
        h = jnp.maximum(
            jnp.dot(h, w2_ref[...], preferred_element_type=jnp.float32)
            + b2_ref[...], 0.0)
        mu = jnp.mean(h, axis=0, keepdims=True)
        var = jnp.mean((h - mu) ** 2, axis=0, keepdims=True)
        h = g2_ref[...] * (h - mu) / jnp.sqrt(var + 1e-5) + be2_ref[...]
        o_ref[...] = (
            jnp.dot(h, w3_ref[...], preferred_element_type=jnp.float32)
            + b3_ref[...])

    return pl.pallas_call(
        body,
        out_shape=jax.ShapeDtypeStruct((x.shape[0], w3.shape[1]), jnp.float32),
    )(x, w1, b1, g1, be1, w2, b2, g2, be2, w3, b3)


_SA1_RADII, _SA1_NSAMPLE = [0.1, 0.2, 0.4], [16, 32, 128]
_SA2_RADII, _SA2_NSAMPLE = [0.2, 0.4, 0.8], [32, 64, 128]


def _level_weights(params, prefix, i, c_feat, c_pad):
    w0 = params[f"{prefix}_r{i}_w0"]
    c1 = w0.shape[1]
    w1 = jnp.zeros((c_pad, c1), jnp.float32).at[: c_feat + 3].set(w0)
    b1 = params[f"{prefix}_r{i}_b0"].reshape(1, -1)
    w2 = params[f"{prefix}_r{i}_w1"]
    b2 = params[f"{prefix}_r{i}_b1"].reshape(1, -1)
    w3 = params[f"{prefix}_r{i}_w2"]
    b3 = params[f"{prefix}_r{i}_b2"].reshape(1, -1)
    return w1, b1, w2, b2, w3, b3


def kernel(xyz, points, params):
    B, N1, _ = xyz.shape
    P1, P2 = 512, 128

    t1 = jnp.concatenate([points, xyz], axis=-1)
    t1 = jnp.pad(t1, ((0, 0), (0, 0), (0, 10))).reshape(B * N1, 16)
    xyzT = jnp.transpose(xyz, (2, 0, 1))

    cent1 = _fps_call(xyzT, P1)
    c1rows = jnp.transpose(cent1, (1, 2, 0)).reshape(B * P1, 3)
    c1cols = jnp.pad(c1rows, ((0, 0), (3, 10)))

    groupeds = _ball_gather_msg(xyzT, cent1, t1, _SA1_RADII, _SA1_NSAMPLE,
                                P1, N1, 16)
    parts = []
    for i, K in enumerate(_SA1_NSAMPLE):
        w1, b1, w2, b2, w3, b3 = _level_weights(params, "sa1", i, 3, 16)
        parts.append(_mlp_max_call(groupeds[i], c1cols, w1, b1, w2, b2,
                                   w3, b3, K, CB=64))
    feat1 = jnp.concatenate(parts, axis=-1)

    t2 = jnp.pad(jnp.concatenate([feat1, c1rows], axis=-1), ((0, 0), (0, 13)))

    cent2 = _fps_call(cent1, P2)
    c2rows = jnp.transpose(cent2, (1, 2, 0)).reshape(B * P2, 3)
    c2cols = jnp.pad(c2rows, ((0, 0), (320, 13)))

    groupeds = _ball_gather_msg(cent1, cent2, t2, _SA2_RADII, _SA2_NSAMPLE,
                                P2, P1, 336)
    parts = []
    for i, K in enumerate(_SA2_NSAMPLE):
        w1, b1, w2, b2, w3, b3 = _level_weights(params, "sa2", i, 320, 336)
        parts.append(_mlp_max_call(groupeds[i], c2cols, w1, b1, w2, b2,
                                   w3, b3, K, CB=16))
    feat2 = jnp.concatenate(parts, axis=-1)

    t3 = jnp.concatenate([feat2, c2rows], axis=-1)
    feat3 = _sa3_call(
        t3,
        params["sa3_w0"], params["sa3_b0"].reshape(1, -1),
        params["sa3_w1"], params["sa3_b1"].reshape(1, -1),
        params["sa3_w2"], params["sa3_b2"].reshape(1, -1),
        P2)

    return _head_call(
        feat3,
        params["dn1_w"], params["dn1_b"].reshape(1, -1),
        params["bn1_gamma"].reshape(1, -1), params["bn1_beta"].reshape(1, -1),
        params["dn2_w"], params["dn2_b"].reshape(1, -1),
        params["bn2_gamma"].reshape(1, -1), params["bn2_beta"].reshape(1, -1),
        params["dn3_w"], params["dn3_b"].reshape(1, -1))


# speedup vs baseline: 1.5639x; 1.0016x over previous
import functools

import jax
import jax.numpy as jnp
from jax import lax
from jax.experimental import pallas as pl
from jax.experimental.pallas import tpu as pltpu
from jax.experimental.pallas import tpu_sc as plsc

_B = 8
_NW = 32


def _fps_call(xyzT, npoint):
    _, b, n = xyzT.shape

    def body(xyzT_ref, cent_ref):
        x = xyzT_ref[0]
        y = xyzT_ref[1]
        z = xyzT_ref[2]
        iota_n = lax.broadcasted_iota(jnp.int32, (b, n), 1)

        iota_c = lax.broadcasted_iota(jnp.int32, (b, 128), 1)

        def step(i, carry):
            dists, far, ax, ay, az = carry
            oh = iota_n == far
            cx = jnp.sum(jnp.where(oh, x, 0.0), axis=1, keepdims=True)
            cy = jnp.sum(jnp.where(oh, y, 0.0), axis=1, keepdims=True)
            cz = jnp.sum(jnp.where(oh, z, 0.0), axis=1, keepdims=True)
            sel = iota_c == lax.rem(i, 128)
            ax = jnp.where(sel, cx, ax)
            ay = jnp.where(sel, cy, ay)
            az = jnp.where(sel, cz, az)
            dx = x - cx
            dy = y - cy
            dz = z - cz
            d = dx * dx + dy * dy + dz * dz
            dists = jnp.minimum(dists, d)
            m = jnp.max(dists, axis=1, keepdims=True)
            cand = jnp.where(dists == m, iota_n, n)
            far = jnp.min(cand, axis=1, keepdims=True)
            return dists, far, ax, ay, az

        carry = (
            jnp.full((b, n), 1e10, jnp.float32),
            jnp.zeros((b, 1), jnp.int32),
            jnp.zeros((b, 128), jnp.float32),
            jnp.zeros((b, 128), jnp.float32),
            jnp.zeros((b, 128), jnp.float32),
        )
        for blk in range(npoint // 128):
            carry = lax.fori_loop(blk * 128, (blk + 1) * 128, step, carry)
            cent_ref[0, :, blk * 128:(blk + 1) * 128] = carry[2]
            cent_ref[1, :, blk * 128:(blk + 1) * 128] = carry[3]
            cent_ref[2, :, blk * 128:(blk + 1) * 128] = carry[4]

    return pl.pallas_call(
        body,
        out_shape=jax.ShapeDtypeStruct((3, b, npoint), jnp.float32),
    )(xyzT)


def _ball_gather_msg(xyzT, centT, table, radii, Ks, P, N, C):
    tpb = _NW // _B
    PB = P // tpb
    nchunks = N // 16
    NR = len(Ks)
    r2s = [r * r for r in radii]
    Gs = [max(1, 128 // K) for K in Ks]
    GKs = [G * K for G, K in zip(Gs, Ks)]
    mesh = plsc.VectorSubcoreMesh(core_axis_name="c", subcore_axis_name="s")

    @functools.partial(
        pl.kernel,
        mesh=mesh,
        compiler_params=pltpu.CompilerParams(needs_layout_passes=False,
                                             use_tc_tiling_on_sc=False),
        out_type=tuple(jax.ShapeDtypeStruct((_B * P * K, C), jnp.float32)
                       for K in Ks),
        scratch_types=[
            pltpu.VMEM((N,), jnp.float32),
            pltpu.VMEM((N,), jnp.float32),
            pltpu.VMEM((N,), jnp.float32),
            pltpu.VMEM((PB,), jnp.float32),
            pltpu.VMEM((PB,), jnp.float32),
            pltpu.VMEM((PB,), jnp.float32),
            *[pltpu.VMEM((K + 16,), jnp.int32) for K in Ks],
            *[pltpu.VMEM((PB * K,), jnp.int32) for K in Ks],
            pltpu.VMEM((max(GKs), C), jnp.float32),
            pltpu.VMEM((max(GKs), C), jnp.float32),
            pltpu.SemaphoreType.DMA,
            pltpu.SemaphoreType.DMA,
        ],
    )
    def bq(xyzT_hbm, centT_hbm, table_hbm, *refs):
        outs = refs[:NR]
        xb, yb, zb, cxb, cyb, czb = refs[NR:NR + 6]
        raws = refs[NR + 6:NR + 6 + NR]
        idxs = refs[NR + 6 + NR:NR + 6 + 2 * NR]
        rows0, rows1, sem0, sem1 = refs[NR + 6 + 2 * NR:]
        wid = lax.axis_index("s") * 2 + lax.axis_index("c")
        bi = wid // tpb
        p0 = (wid % tpb) * PB
        pltpu.sync_copy(xyzT_hbm.at[0, bi], xb)
        pltpu.sync_copy(xyzT_hbm.at[1, bi], yb)
        pltpu.sync_copy(xyzT_hbm.at[2, bi], zb)
        pltpu.sync_copy(centT_hbm.at[0, bi, pl.ds(p0, PB)], cxb)
        pltpu.sync_copy(centT_hbm.at[1, bi, pl.ds(p0, PB)], cyb)
        pltpu.sync_copy(centT_hbm.at[2, bi, pl.ds(p0, PB)], czb)
        lane = lax.iota(jnp.int32, 16)
        base_row = bi * N

        def per_chunk(cc, carry):
            cxv = cxb[pl.ds(cc * 16, 16)]
            cyv = cyb[pl.ds(cc * 16, 16)]
            czv = czb[pl.ds(cc * 16, 16)]
            for j in range(16):
                cx = cxv[j]
                cy = cyv[j]
                cz = czv[j]

                def chunk(ch, cnts):
                    off = ch * 16
                    dx = xb[pl.ds(off, 16)] - cx
                    dy = yb[pl.ds(off, 16)] - cy
                    dz = zb[pl.ds(off, 16)] - cz
                    d = dx * dx + dy * dy + dz * dz
                    new = []
                    for i in range(NR):
                        m = d <= r2s[i]
                        key = jnp.where(m, lane + off, 0x40000000 + lane)
                        ks, _ = plsc.sort_key_val(key, key)
                        raws[i][pl.ds(jnp.minimum(cnts[i], Ks[i]), 16)] = ks
                        new.append(
                            cnts[i] + plsc.all_reduce_population_count(m)[0])
                    return tuple(new)

                cnts = lax.fori_loop(0, nchunks, chunk,
                                     (jnp.int32(0),) * NR)
                ci = cc * 16 + j
                for i in range(NR):
                    first = raws[i][pl.ds(0, 16)][0]
                    for q in range(Ks[i] // 16):
                        v = raws[i][pl.ds(q * 16, 16)]
                        v = jnp.where(lane + (q * 16) < cnts[i], v, first)
                        idxs[i][pl.ds(ci * Ks[i] + q * 16, 16)] = v + base_row
            return carry

        lax.fori_loop(0, PB // 16, per_chunk, 0)

        for i in range(NR):
            K, GK = Ks[i], GKs[i]
            ngroups = PB // Gs[i]
            idx_all = idxs[i]
            out_hbm = outs[i]
            out_base = (bi * P + p0) * K

            def fire(g, rows, sem):
                pltpu.async_copy(
                    table_hbm.at[idx_all.at[pl.ds(g * GK, GK)]],
                    rows.at[pl.ds(0, GK)], sem)

            def drain(g, rows, sem):
                pltpu.make_async_copy(
                    table_hbm.at[idx_all.at[pl.ds(g * GK, GK)]],
                    rows.at[pl.ds(0, GK)], sem).wait()
                pltpu.sync_copy(rows.at[pl.ds(0, GK)],
                                out_hbm.at[pl.ds(out_base + g * GK, GK)])

            def gloop(g, carry):
                even = g % 2 == 0

                @pl.when(even)
                def _():
                    fire(g, rows0, sem0)

                @pl.when(jnp.logical_not(even))
                def _():
                    fire(g, rows1, sem1)

                @pl.when(jnp.logical_and(g > 0, even))
                def _():
                    drain(g - 1, rows1, sem1)

                @pl.when(jnp.logical_and(g > 0, jnp.logical_not(even)))
                def _():
                    drain(g - 1, rows0, sem0)
                return carry

            lax.fori_loop(0, ngroups, gloop, 0)
            if ngroups % 2 == 1:
                drain(ngroups - 1, rows0, sem0)
            else:
                drain(ngroups - 1, rows1, sem1)

    return bq(xyzT, centT, table)


def _mlp_max_call(grouped, cent_cols, w1, b1, w2, b2, w3, b3, K, CB):
    _, C = grouped.shape
    BP = cent_cols.shape[0]
    C1 = w1.shape[1]
    C3 = w3.shape[1]

    def body(g_ref, c_ref, w1_ref, b1_ref, w2_ref, b2_ref,
             w3_ref, b3_ref, o_ref):
        X = g_ref[...].reshape(CB, K, C) - c_ref[...][:, None, :]
        h = jnp.dot(X.reshape(CB * K, C), w1_ref[...],
                    preferred_element_type=jnp.float32)
        h = jnp.maximum(h + b1_ref[...], 0.0)
        h = jnp.maximum(
            jnp.dot(h, w2_ref[...], preferred_element_type=jnp.float32)
            + b2_ref[...], 0.0)
        h = jnp.maximum(
            jnp.dot(h, w3_ref[...], preferred_element_type=jnp.float32)
            + b3_ref[...], 0.0)
        o_ref[...] = jnp.max(h.reshape(CB, K, C3), axis=1)

    rep = lambda shape: pl.BlockSpec(shape, lambda i: (0, 0))
    return pl.pallas_call(
        body,
        grid=(BP // CB,),
        in_specs=[
            pl.BlockSpec((CB * K, C), lambda i: (i, 0)),
            pl.BlockSpec((CB, C), lambda i: (i, 0)),
            rep(w1.shape), rep(b1.shape),
            rep(w2.shape), rep(b2.shape), rep(w3.shape), rep(b3.shape),
        ],
        out_specs=pl.BlockSpec((CB, C3), lambda i: (i, 0)),
        out_shape=jax.ShapeDtypeStruct((BP, C3), jnp.float32),
    )(grouped, cent_cols, w1, b1, w2, b2, w3, b3)


def _sa3_call(tbl, w1, b1, w2, b2, w3, b3, P):
    _, C = tbl.shape
    C3 = w3.shape[1]

    def body(x_ref, w1_ref, b1_ref, w2_ref, b2_ref, w3_ref, b3_ref, o_ref):
        h = jnp.maximum(
            jnp.dot(x_ref[...], w1_ref[...], preferred_element_type=jnp.float32)
            + b1_ref[...], 0.0)
        h = jnp.maximum(
            jnp.dot(h, w2_ref[...], preferred_element_type=jnp.float32)
            + b2_ref[...], 0.0)
        h = jnp.maximum(
            jnp.dot(h, w3_ref[...], preferred_element_type=jnp.float32)
            + b3_ref[...], 0.0)
        o_ref[...] = jnp.max(h, axis=0, keepdims=True)[None]

    rep = lambda shape: pl.BlockSpec(shape, lambda i: (0, 0))
    out = pl.pallas_call(
        body,
        grid=(_B,),
        in_specs=[
            pl.BlockSpec((P, C), lambda i: (i, 0)),
            rep(w1.shape), rep(b1.shape), rep(w2.shape), rep(b2.shape),
            rep(w3.shape), rep(b3.shape),
        ],
        out_specs=pl.BlockSpec((1, 1, C3), lambda i: (i, 0, 0)),
        out_shape=jax.ShapeDtypeStruct((_B, 1, C3), jnp.float32),
    )(tbl, w1, b1, w2, b2, w3, b3)
    return out.reshape(_B, C3)


def _head_call(x, w1, b1, g1, be1, w2, b2, g2, be2, w3, b3):
    def body(x_ref, w1_ref, b1_ref, g1_ref, be1_ref, w2_ref, b2_ref,
             g2_ref, be2_ref, w3_ref, b3_ref, o_ref):
        h = jnp.maximum(
            jnp.dot(x_ref[...], w1_ref[...], preferred_element_type=jnp.float32)
            + b1_ref[...], 0.0)
        mu = jnp.mean(h, axis=0, keepdims=True)
        var = jnp.mean((h - mu) ** 2, axis=0, keepdims=True)
        h = g1_ref[...] * (h - mu) / jnp.sqrt(var + 1e-5) + be1_ref[...]
        h = jnp.maximum(
            jnp.dot(h, w2_ref[...], preferred_element_type=jnp.float32)
            + b2_ref[...], 0.0)
        mu = jnp.mean(h, axis=0, keepdims=True)
        var = jnp.mean((h - mu) ** 2, axis=0, keepdims=True)
        h = g2_ref[...] * (h - mu) / jnp.sqrt(var + 1e-5) + be2_ref[...]
        o_ref[...] = (
            jnp.dot(h, w3_ref[...], preferred_element_type=jnp.float32)
            + b3_ref[...])

    return pl.pallas_call(
        body,
        out_shape=jax.ShapeDtypeStruct((x.shape[0], w3.shape[1]), jnp.float32),
    )(x, w1, b1, g1, be1, w2, b2, g2, be2, w3, b3)


_SA1_RADII, _SA1_NSAMPLE = [0.1, 0.2, 0.4], [16, 32, 128]
_SA2_RADII, _SA2_NSAMPLE = [0.2, 0.4, 0.8], [32, 64, 128]


def _level_weights(params, prefix, i, c_feat, c_pad):
    w0 = params[f"{prefix}_r{i}_w0"]
    c1 = w0.shape[1]
    w1 = jnp.zeros((c_pad, c1), jnp.float32).at[: c_feat + 3].set(w0)
    b1 = params[f"{prefix}_r{i}_b0"].reshape(1, -1)
    w2 = params[f"{prefix}_r{i}_w1"]
    b2 = params[f"{prefix}_r{i}_b1"].reshape(1, -1)
    w3 = params[f"{prefix}_r{i}_w2"]
    b3 = params[f"{prefix}_r{i}_b2"].reshape(1, -1)
    return w1, b1, w2, b2, w3, b3


def kernel(xyz, points, params):
    B, N1, _ = xyz.shape
    P1, P2 = 512, 128

    t1 = jnp.concatenate([points, xyz], axis=-1)
    t1 = jnp.pad(t1, ((0, 0), (0, 0), (0, 10))).reshape(B * N1, 16)
    xyzT = jnp.transpose(xyz, (2, 0, 1))

    cent1 = _fps_call(xyzT, P1)
    c1rows = jnp.transpose(cent1, (1, 2, 0)).reshape(B * P1, 3)
    c1cols = jnp.pad(c1rows, ((0, 0), (3, 10)))

    groupeds = _ball_gather_msg(xyzT, cent1, t1, _SA1_RADII, _SA1_NSAMPLE,
                                P1, N1, 16)
    parts = []
    for i, K in enumerate(_SA1_NSAMPLE):
        w1, b1, w2, b2, w3, b3 = _level_weights(params, "sa1", i, 3, 16)
        parts.append(_mlp_max_call(groupeds[i], c1cols, w1, b1, w2, b2,
                                   w3, b3, K, CB=64))
    feat1 = jnp.concatenate(parts, axis=-1)

    t2 = jnp.pad(jnp.concatenate([feat1, c1rows], axis=-1), ((0, 0), (0, 13)))

    cent2 = _fps_call(cent1, P2)
    c2rows = jnp.transpose(cent2, (1, 2, 0)).reshape(B * P2, 3)
    c2cols = jnp.pad(c2rows, ((0, 0), (320, 13)))

    groupeds = _ball_gather_msg(cent1, cent2, t2, _SA2_RADII, _SA2_NSAMPLE,
                                P2, P1, 336)
    parts = []
    for i, K in enumerate(_SA2_NSAMPLE):
        w1, b1, w2, b2, w3, b3 = _level_weights(params, "sa2", i, 320, 336)
        parts.append(_mlp_max_call(groupeds[i], c2cols, w1, b1, w2, b2,
                                   w3, b3, K, CB=16))
    feat2 = jnp.concatenate(parts, axis=-1)

    t3 = jnp.concatenate([feat2, c2rows], axis=-1)
    feat3 = _sa3_call(
        t3,
        params["sa3_w0"], params["sa3_b0"].reshape(1, -1),
        params["sa3_w1"], params["sa3_b1"].reshape(1, -1),
        params["sa3_w2"], params["sa3_b2"].reshape(1, -1),
        P2)

    return _head_call(
        feat3,
        params["dn1_w"], params["dn1_b"].reshape(1, -1),
        params["bn1_gamma"].reshape(1, -1), params["bn1_beta"].reshape(1, -1),
        params["dn2_w"], params["dn2_b"].reshape(1, -1),
        params["bn2_gamma"].reshape(1, -1), params["bn2_beta"].reshape(1, -1),
        params["dn3_w"], params["dn3_b"].reshape(1, -1))


# speedup vs baseline: 1.6896x; 1.0804x over previous
import functools

import jax
import jax.numpy as jnp
from jax import lax
from jax.experimental import pallas as pl
from jax.experimental.pallas import tpu as pltpu
from jax.experimental.pallas import tpu_sc as plsc

_B = 8
_NW = 32


def _fps_call(xyzT, npoint):
    _, b, n = xyzT.shape

    def body(xyzT_ref, cent_ref):
        x = xyzT_ref[0]
        y = xyzT_ref[1]
        z = xyzT_ref[2]
        iota_n = lax.broadcasted_iota(jnp.int32, (b, n), 1)

        iota_c = lax.broadcasted_iota(jnp.int32, (b, 128), 1)

        def step(i, carry):
            dists, far, ax, ay, az = carry
            oh = iota_n == far
            cx = jnp.sum(jnp.where(oh, x, 0.0), axis=1, keepdims=True)
            cy = jnp.sum(jnp.where(oh, y, 0.0), axis=1, keepdims=True)
            cz = jnp.sum(jnp.where(oh, z, 0.0), axis=1, keepdims=True)
            sel = iota_c == lax.rem(i, 128)
            ax = jnp.where(sel, cx, ax)
            ay = jnp.where(sel, cy, ay)
            az = jnp.where(sel, cz, az)
            dx = x - cx
            dy = y - cy
            dz = z - cz
            d = dx * dx + dy * dy + dz * dz
            dists = jnp.minimum(dists, d)
            m = jnp.max(dists, axis=1, keepdims=True)
            cand = jnp.where(dists == m, iota_n, n)
            far = jnp.min(cand, axis=1, keepdims=True)
            return dists, far, ax, ay, az

        carry = (
            jnp.full((b, n), 1e10, jnp.float32),
            jnp.zeros((b, 1), jnp.int32),
            jnp.zeros((b, 128), jnp.float32),
            jnp.zeros((b, 128), jnp.float32),
            jnp.zeros((b, 128), jnp.float32),
        )
        for blk in range(npoint // 128):
            carry = lax.fori_loop(blk * 128, (blk + 1) * 128, step, carry)
            cent_ref[0, :, blk * 128:(blk + 1) * 128] = carry[2]
            cent_ref[1, :, blk * 128:(blk + 1) * 128] = carry[3]
            cent_ref[2, :, blk * 128:(blk + 1) * 128] = carry[4]

    return pl.pallas_call(
        body,
        out_shape=jax.ShapeDtypeStruct((3, b, npoint), jnp.float32),
    )(xyzT)


def _ball_gather_msg(xyzT, centT, table, radii, Ks, P, N, C):
    tpb = _NW // _B
    PB = P // tpb
    nchunks = N // 16
    NR = len(Ks)
    r2s = [r * r for r in radii]
    Gs = [max(1, 128 // K) for K in Ks]
    GKs = [G * K for G, K in zip(Gs, Ks)]
    mesh = plsc.VectorSubcoreMesh(core_axis_name="c", subcore_axis_name="s")

    @functools.partial(
        pl.kernel,
        mesh=mesh,
        compiler_params=pltpu.CompilerParams(needs_layout_passes=False,
                                             use_tc_tiling_on_sc=False),
        out_type=tuple(jax.ShapeDtypeStruct((_B * P * K, C), jnp.float32)
                       for K in Ks),
        scratch_types=[
            pltpu.VMEM((N,), jnp.float32),
            pltpu.VMEM((N,), jnp.float32),
            pltpu.VMEM((N,), jnp.float32),
            pltpu.VMEM((PB,), jnp.float32),
            pltpu.VMEM((PB,), jnp.float32),
            pltpu.VMEM((PB,), jnp.float32),
            *[pltpu.VMEM((K + 16,), jnp.int32) for K in Ks],
            *[pltpu.VMEM((K + 16,), jnp.int32) for K in Ks],
            *[pltpu.VMEM((PB * K,), jnp.int32) for K in Ks],
            pltpu.VMEM((max(GKs), C), jnp.float32),
            pltpu.VMEM((max(GKs), C), jnp.float32),
            pltpu.SemaphoreType.DMA,
            pltpu.SemaphoreType.DMA,
        ],
    )
    def bq(xyzT_hbm, centT_hbm, table_hbm, *refs):
        outs = refs[:NR]
        xb, yb, zb, cxb, cyb, czb = refs[NR:NR + 6]
        raws_ab = (refs[NR + 6:NR + 6 + NR], refs[NR + 6 + NR:NR + 6 + 2 * NR])
        idxs = refs[NR + 6 + 2 * NR:NR + 6 + 3 * NR]
        rows0, rows1, sem0, sem1 = refs[NR + 6 + 3 * NR:]
        wid = lax.axis_index("s") * 2 + lax.axis_index("c")
        bi = wid // tpb
        p0 = (wid % tpb) * PB
        pltpu.sync_copy(xyzT_hbm.at[0, bi], xb)
        pltpu.sync_copy(xyzT_hbm.at[1, bi], yb)
        pltpu.sync_copy(xyzT_hbm.at[2, bi], zb)
        pltpu.sync_copy(centT_hbm.at[0, bi, pl.ds(p0, PB)], cxb)
        pltpu.sync_copy(centT_hbm.at[1, bi, pl.ds(p0, PB)], cyb)
        pltpu.sync_copy(centT_hbm.at[2, bi, pl.ds(p0, PB)], czb)
        lane = lax.iota(jnp.int32, 16)
        base_row = bi * N

        def per_chunk(cc, carry):
            cxv = cxb[pl.ds(cc * 16, 16)]
            cyv = cyb[pl.ds(cc * 16, 16)]
            czv = czb[pl.ds(cc * 16, 16)]
            for j in range(0, 16, 2):
                cab = [(cxv[j], cyv[j], czv[j]),
                       (cxv[j + 1], cyv[j + 1], czv[j + 1])]

                def chunk(ch, cnts):
                    off = ch * 16
                    xv = xb[pl.ds(off, 16)]
                    yv = yb[pl.ds(off, 16)]
                    zv = zb[pl.ds(off, 16)]
                    new = []
                    for a in range(2):
                        cx, cy, cz = cab[a]
                        dx = xv - cx
                        dy = yv - cy
                        dz = zv - cz
                        d = dx * dx + dy * dy + dz * dz
                        for i in range(NR):
                            m = d <= r2s[i]
                            key = jnp.where(m, lane + off,
                                            0x40000000 + lane)
                            ks, _ = plsc.sort_key_val(key, key)
                            cnt = cnts[a * NR + i]
                            raws_ab[a][i][
                                pl.ds(jnp.minimum(cnt, Ks[i]), 16)] = ks
                            new.append(
                                cnt
                                + plsc.all_reduce_population_count(m)[0])
                    return tuple(new)

                cnts = lax.fori_loop(0, nchunks, chunk,
                                     (jnp.int32(0),) * (2 * NR))
                for a in range(2):
                    ci = cc * 16 + j + a
                    for i in range(NR):
                        first = raws_ab[a][i][pl.ds(0, 16)][0]
                        cnt = cnts[a * NR + i]
                        for q in range(Ks[i] // 16):
                            v = raws_ab[a][i][pl.ds(q * 16, 16)]
                            v = jnp.where(lane + (q * 16) < cnt, v, first)
                            idxs[i][pl.ds(ci * Ks[i] + q * 16, 16)] = (
                                v + base_row)
            return carry

        lax.fori_loop(0, PB // 16, per_chunk, 0)

        for i in range(NR):
            K, GK = Ks[i], GKs[i]
            ngroups = PB // Gs[i]
            idx_all = idxs[i]
            out_hbm = outs[i]
            out_base = (bi * P + p0) * K

            def fire(g, rows, sem):
                pltpu.async_copy(
                    table_hbm.at[idx_all.at[pl.ds(g * GK, GK)]],
                    rows.at[pl.ds(0, GK)], sem)

            def drain(g, rows, sem):
                pltpu.make_async_copy(
                    table_hbm.at[idx_all.at[pl.ds(g * GK, GK)]],
                    rows.at[pl.ds(0, GK)], sem).wait()
                pltpu.sync_copy(rows.at[pl.ds(0, GK)],
                                out_hbm.at[pl.ds(out_base + g * GK, GK)])

            def gloop(g, carry):
                even = g % 2 == 0

                @pl.when(even)
                def _():
                    fire(g, rows0, sem0)

                @pl.when(jnp.logical_not(even))
                def _():
                    fire(g, rows1, sem1)

                @pl.when(jnp.logical_and(g > 0, even))
                def _():
                    drain(g - 1, rows1, sem1)

                @pl.when(jnp.logical_and(g > 0, jnp.logical_not(even)))
                def _():
                    drain(g - 1, rows0, sem0)
                return carry

            lax.fori_loop(0, ngroups, gloop, 0)
            if ngroups % 2 == 1:
                drain(ngroups - 1, rows0, sem0)
            else:
                drain(ngroups - 1, rows1, sem1)

    return bq(xyzT, centT, table)


def _mlp_max_call(grouped, cent_cols, w1, b1, w2, b2, w3, b3, K, CB):
    _, C = grouped.shape
    BP = cent_cols.shape[0]
    C1 = w1.shape[1]
    C3 = w3.shape[1]

    def body(g_ref, c_ref, w1_ref, b1_ref, w2_ref, b2_ref,
             w3_ref, b3_ref, o_ref):
        X = g_ref[...].reshape(CB, K, C) - c_ref[...][:, None, :]
        h = jnp.dot(X.reshape(CB * K, C), w1_ref[...],
                    preferred_element_type=jnp.float32)
        h = jnp.maximum(h + b1_ref[...], 0.0)
        h = jnp.maximum(
            jnp.dot(h, w2_ref[...], preferred_element_type=jnp.float32)
            + b2_ref[...], 0.0)
        h = jnp.maximum(
            jnp.dot(h, w3_ref[...], preferred_element_type=jnp.float32)
            + b3_ref[...], 0.0)
        o_ref[...] = jnp.max(h.reshape(CB, K, C3), axis=1)

    rep = lambda shape: pl.BlockSpec(shape, lambda i: (0, 0))
    return pl.pallas_call(
        body,
        grid=(BP // CB,),
        in_specs=[
            pl.BlockSpec((CB * K, C), lambda i: (i, 0)),
            pl.BlockSpec((CB, C), lambda i: (i, 0)),
            rep(w1.shape), rep(b1.shape),
            rep(w2.shape), rep(b2.shape), rep(w3.shape), rep(b3.shape),
        ],
        out_specs=pl.BlockSpec((CB, C3), lambda i: (i, 0)),
        out_shape=jax.ShapeDtypeStruct((BP, C3), jnp.float32),
    )(grouped, cent_cols, w1, b1, w2, b2, w3, b3)


def _sa3_call(tbl, w1, b1, w2, b2, w3, b3, P):
    _, C = tbl.shape
    C3 = w3.shape[1]

    def body(x_ref, w1_ref, b1_ref, w2_ref, b2_ref, w3_ref, b3_ref, o_ref):
        h = jnp.maximum(
            jnp.dot(x_ref[...], w1_ref[...], preferred_element_type=jnp.float32)
            + b1_ref[...], 0.0)
        h = jnp.maximum(
            jnp.dot(h, w2_ref[...], preferred_element_type=jnp.float32)
            + b2_ref[...], 0.0)
        h = jnp.maximum(
            jnp.dot(h, w3_ref[...], preferred_element_type=jnp.float32)
            + b3_ref[...], 0.0)
        o_ref[...] = jnp.max(h, axis=0, keepdims=True)[None]

    rep = lambda shape: pl.BlockSpec(shape, lambda i: (0, 0))
    out = pl.pallas_call(
        body,
        grid=(_B,),
        in_specs=[
            pl.BlockSpec((P, C), lambda i: (i, 0)),
            rep(w1.shape), rep(b1.shape), rep(w2.shape), rep(b2.shape),
            rep(w3.shape), rep(b3.shape),
        ],
        out_specs=pl.BlockSpec((1, 1, C3), lambda i: (i, 0, 0)),
        out_shape=jax.ShapeDtypeStruct((_B, 1, C3), jnp.float32),
    )(tbl, w1, b1, w2, b2, w3, b3)
    return out.reshape(_B, C3)


def _head_call(x, w1, b1, g1, be1, w2, b2, g2, be2, w3, b3):
    def body(x_ref, w1_ref, b1_ref, g1_ref, be1_ref, w2_ref, b2_ref,
             g2_ref, be2_ref, w3_ref, b3_ref, o_ref):
        h = jnp.maximum(
            jnp.dot(x_ref[...], w1_ref[...], preferred_element_type=jnp.float32)
            + b1_ref[...], 0.0)
        mu = jnp.mean(h, axis=0, keepdims=True)
        var = jnp.mean((h - mu) ** 2, axis=0, keepdims=True)
        h = g1_ref[...] * (h - mu) / jnp.sqrt(var + 1e-5) + be1_ref[...]
        h = jnp.maximum(
            jnp.dot(h, w2_ref[...], preferred_element_type=jnp.float32)
            + b2_ref[...], 0.0)
        mu = jnp.mean(h, axis=0, keepdims=True)
        var = jnp.mean((h - mu) ** 2, axis=0, keepdims=True)
        h = g2_ref[...] * (h - mu) / jnp.sqrt(var + 1e-5) + be2_ref[...]
        o_ref[...] = (
            jnp.dot(h, w3_ref[...], preferred_element_type=jnp.float32)
            + b3_ref[...])

    return pl.pallas_call(
        body,
        out_shape=jax.ShapeDtypeStruct((x.shape[0], w3.shape[1]), jnp.float32),
    )(x, w1, b1, g1, be1, w2, b2, g2, be2, w3, b3)


_SA1_RADII, _SA1_NSAMPLE = [0.1, 0.2, 0.4], [16, 32, 128]
_SA2_RADII, _SA2_NSAMPLE = [0.2, 0.4, 0.8], [32, 64, 128]


def _level_weights(params, prefix, i, c_feat, c_pad):
    w0 = params[f"{prefix}_r{i}_w0"]
    c1 = w0.shape[1]
    w1 = jnp.zeros((c_pad, c1), jnp.float32).at[: c_feat + 3].set(w0)
    b1 = params[f"{prefix}_r{i}_b0"].reshape(1, -1)
    w2 = params[f"{prefix}_r{i}_w1"]
    b2 = params[f"{prefix}_r{i}_b1"].reshape(1, -1)
    w3 = params[f"{prefix}_r{i}_w2"]
    b3 = params[f"{prefix}_r{i}_b2"].reshape(1, -1)
    return w1, b1, w2, b2, w3, b3


def kernel(xyz, points, params):
    B, N1, _ = xyz.shape
    P1, P2 = 512, 128

    t1 = jnp.concatenate([points, xyz], axis=-1)
    t1 = jnp.pad(t1, ((0, 0), (0, 0), (0, 10))).reshape(B * N1, 16)
    xyzT = jnp.transpose(xyz, (2, 0, 1))

    cent1 = _fps_call(xyzT, P1)
    c1rows = jnp.transpose(cent1, (1, 2, 0)).reshape(B * P1, 3)
    c1cols = jnp.pad(c1rows, ((0, 0), (3, 10)))

    groupeds = _ball_gather_msg(xyzT, cent1, t1, _SA1_RADII, _SA1_NSAMPLE,
                                P1, N1, 16)
    parts = []
    for i, K in enumerate(_SA1_NSAMPLE):
        w1, b1, w2, b2, w3, b3 = _level_weights(params, "sa1", i, 3, 16)
        parts.append(_mlp_max_call(groupeds[i], c1cols, w1, b1, w2, b2,
                                   w3, b3, K, CB=64))
    feat1 = jnp.concatenate(parts, axis=-1)

    t2 = jnp.pad(jnp.concatenate([feat1, c1rows], axis=-1), ((0, 0), (0, 13)))

    cent2 = _fps_call(cent1, P2)
    c2rows = jnp.transpose(cent2, (1, 2, 0)).reshape(B * P2, 3)
    c2cols = jnp.pad(c2rows, ((0, 0), (320, 13)))

    groupeds = _ball_gather_msg(cent1, cent2, t2, _SA2_RADII, _SA2_NSAMPLE,
                                P2, P1, 336)
    parts = []
    for i, K in enumerate(_SA2_NSAMPLE):
        w1, b1, w2, b2, w3, b3 = _level_weights(params, "sa2", i, 320, 336)
        parts.append(_mlp_max_call(groupeds[i], c2cols, w1, b1, w2, b2,
                                   w3, b3, K, CB=16))
    feat2 = jnp.concatenate(parts, axis=-1)

    t3 = jnp.concatenate([feat2, c2rows], axis=-1)
    feat3 = _sa3_call(
        t3,
        params["sa3_w0"], params["sa3_b0"].reshape(1, -1),
        params["sa3_w1"], params["sa3_b1"].reshape(1, -1),
        params["sa3_w2"], params["sa3_b2"].reshape(1, -1),
        P2)

    return _head_call(
        feat3,
        params["dn1_w"], params["dn1_b"].reshape(1, -1),
        params["bn1_gamma"].reshape(1, -1), params["bn1_beta"].reshape(1, -1),
        params["dn2_w"], params["dn2_b"].reshape(1, -1),
        params["bn2_gamma"].reshape(1, -1), params["bn2_beta"].reshape(1, -1),
        params["dn3_w"], params["dn3_b"].reshape(1, -1))
